# baseline (speedup 1.0000x reference)
import functools

import jax
import jax.numpy as jnp
from jax.experimental import pallas as pl
from jax.experimental.pallas import tpu as pltpu


def _chamfer_tile_kernel(xa_ref, ya_ref, out_ref, rowmin_ref, *, inv_bn, inv_bm):
    b = pl.program_id(0)
    m = pl.program_id(1)
    nm = pl.num_programs(1)

    xa = xa_ref[0]
    ya = ya_ref[0]
    d = jax.lax.dot_general(
        xa, ya, (((1,), (1,)), ((), ())), preferred_element_type=jnp.float32
    )

    tile_rowmin = jnp.min(d, axis=1, keepdims=True)

    @pl.when(m == 0)
    def _():
        rowmin_ref[...] = tile_rowmin

    @pl.when(m > 0)
    def _():
        rowmin_ref[...] = jnp.minimum(rowmin_ref[...], tile_rowmin)

    colmin = jnp.min(d, axis=0)
    yx_part = jnp.sum(jnp.maximum(colmin, 0.0)) * inv_bm

    @pl.when((b == 0) & (m == 0))
    def _():
        out_ref[0, 0] = 0.0

    out_ref[0, 0] += yx_part

    @pl.when(m == nm - 1)
    def _():
        xy_sum = jnp.sum(jnp.maximum(rowmin_ref[...], 0.0))
        out_ref[0, 0] += xy_sum * inv_bn


@jax.jit
def kernel(x, y):
    B, N, D = x.shape
    _, M, _ = y.shape
    f32 = jnp.float32

    x = x.astype(f32)
    y = y.astype(f32)
    x2 = jnp.sum(x * x, axis=-1, keepdims=True)
    y2 = jnp.sum(y * y, axis=-1, keepdims=True)
    ones_x = jnp.ones_like(x2)
    ones_y = jnp.ones_like(y2)
    zpad_x = jnp.zeros((B, N, 3), f32)
    zpad_y = jnp.zeros((B, M, 3), f32)
    xa = jnp.concatenate([-2.0 * x, x2, ones_x, zpad_x], axis=-1)
    ya = jnp.concatenate([y, ones_y, y2, zpad_y], axis=-1)

    TM = 1024
    grid = (B, M // TM)

    out = pl.pallas_call(
        functools.partial(
            _chamfer_tile_kernel, inv_bn=1.0 / (B * N), inv_bm=1.0 / (B * M)
        ),
        grid=grid,
        in_specs=[
            pl.BlockSpec((1, N, 8), lambda b, m: (b, 0, 0)),
            pl.BlockSpec((1, TM, 8), lambda b, m: (b, m, 0)),
        ],
        out_specs=pl.BlockSpec(
            (1, 1), lambda b, m: (0, 0), memory_space=pltpu.SMEM
        ),
        out_shape=jax.ShapeDtypeStruct((1, 1), f32),
        scratch_shapes=[pltpu.VMEM((N, 1), f32)],
    )(xa, ya)
    return out[0, 0]


# speedup vs baseline: 1.6768x; 1.6768x over previous
import functools

import jax
import jax.numpy as jnp
from jax.experimental import pallas as pl
from jax.experimental.pallas import tpu as pltpu


def _chamfer_tile_kernel(xa_ref, ya_ref, out_ref, rowmin_ref, *, inv_bn, inv_bm):
    m = pl.program_id(1)
    nm = pl.num_programs(1)

    xa = xa_ref[0]
    ya = ya_ref[0]
    d = jax.lax.dot_general(
        xa, ya, (((1,), (1,)), ((), ())), preferred_element_type=jnp.float32
    ).astype(jnp.bfloat16)

    tile_rowmin = jnp.min(d, axis=1, keepdims=True).astype(jnp.float32)

    @pl.when(m == 0)
    def _():
        rowmin_ref[...] = tile_rowmin

    @pl.when(m > 0)
    def _():
        rowmin_ref[...] = jnp.minimum(rowmin_ref[...], tile_rowmin)

    colmin = jnp.min(d, axis=0).astype(jnp.float32)
    yx_part = jnp.sum(jnp.maximum(colmin, 0.0)) * inv_bm

    @pl.when(m == 0)
    def _():
        out_ref[0, 0, 0] = 0.0

    out_ref[0, 0, 0] += yx_part

    @pl.when(m == nm - 1)
    def _():
        xy_sum = jnp.sum(jnp.maximum(rowmin_ref[...], 0.0))
        out_ref[0, 0, 0] += xy_sum * inv_bn


@jax.jit
def kernel(x, y):
    B, N, D = x.shape
    _, M, _ = y.shape
    f32 = jnp.float32

    x = x.astype(f32)
    y = y.astype(f32)
    x2 = jnp.sum(x * x, axis=-1, keepdims=True)
    y2 = jnp.sum(y * y, axis=-1, keepdims=True)
    ones_x = jnp.ones_like(x2)
    ones_y = jnp.ones_like(y2)
    zpad_x = jnp.zeros((B, N, 3), f32)
    zpad_y = jnp.zeros((B, M, 3), f32)
    xa = jnp.concatenate([-2.0 * x, x2, ones_x, zpad_x], axis=-1)
    ya = jnp.concatenate([y, ones_y, y2, zpad_y], axis=-1)

    bf16 = jnp.bfloat16
    xhi = xa.astype(bf16)
    xlo = (xa - xhi.astype(f32)).astype(bf16)
    yhi = ya.astype(bf16)
    ylo = (ya - yhi.astype(f32)).astype(bf16)
    xs = jnp.concatenate([xhi, xhi, xlo], axis=-1)
    ys = jnp.concatenate([yhi, ylo, yhi], axis=-1)

    TM = 1024
    grid = (B, M // TM)

    out = pl.pallas_call(
        functools.partial(
            _chamfer_tile_kernel, inv_bn=1.0 / (B * N), inv_bm=1.0 / (B * M)
        ),
        grid=grid,
        in_specs=[
            pl.BlockSpec((1, N, 24), lambda b, m: (b, 0, 0)),
            pl.BlockSpec((1, TM, 24), lambda b, m: (b, m, 0)),
        ],
        out_specs=pl.BlockSpec(
            (1, 1, 1), lambda b, m: (b, 0, 0), memory_space=pltpu.SMEM
        ),
        out_shape=jax.ShapeDtypeStruct((B, 1, 1), f32),
        scratch_shapes=[pltpu.VMEM((N, 1), f32)],
        compiler_params=pltpu.CompilerParams(
            dimension_semantics=("parallel", "arbitrary")
        ),
    )(xs, ys)
    return jnp.sum(out)


# speedup vs baseline: 2.1875x; 1.3046x over previous
import functools

import jax
import jax.numpy as jnp
from jax.experimental import pallas as pl
from jax.experimental.pallas import tpu as pltpu


def _chamfer_tile_kernel(xa_ref, ya_ref, out_ref, rowmin_ref, *, inv_bn, inv_bm):
    m = pl.program_id(1)
    nm = pl.num_programs(1)

    xa = xa_ref[0]
    ya = ya_ref[0]
    d = jax.lax.dot_general(
        xa, ya, (((1,), (1,)), ((), ())), preferred_element_type=jnp.float32
    ).astype(jnp.bfloat16)

    tile_rowmin = jnp.min(d, axis=1, keepdims=True).astype(jnp.float32)

    @pl.when(m == 0)
    def _():
        rowmin_ref[...] = tile_rowmin

    @pl.when(m > 0)
    def _():
        rowmin_ref[...] = jnp.minimum(rowmin_ref[...], tile_rowmin)

    colmin = jnp.min(d, axis=0).astype(jnp.float32)
    yx_part = jnp.sum(jnp.maximum(colmin, 0.0)) * inv_bm

    @pl.when(m == 0)
    def _():
        out_ref[0, 0, 0] = 0.0

    out_ref[0, 0, 0] += yx_part

    @pl.when(m == nm - 1)
    def _():
        xy_sum = jnp.sum(jnp.maximum(rowmin_ref[...], 0.0))
        out_ref[0, 0, 0] += xy_sum * inv_bn


@jax.jit
def kernel(x, y):
    B, N, D = x.shape
    _, M, _ = y.shape
    f32 = jnp.float32

    x = x.astype(f32)
    y = y.astype(f32)
    x2 = jnp.sum(x * x, axis=-1, keepdims=True)
    y2 = jnp.sum(y * y, axis=-1, keepdims=True)
    ones_x = jnp.ones_like(x2)
    ones_y = jnp.ones_like(y2)
    zpad_x = jnp.zeros((B, N, 3), f32)
    zpad_y = jnp.zeros((B, M, 3), f32)
    xa = jnp.concatenate([-2.0 * x, x2, ones_x, zpad_x], axis=-1)
    ya = jnp.concatenate([y, ones_y, y2, zpad_y], axis=-1)

    bf16 = jnp.bfloat16
    xhi = xa.astype(bf16)
    xlo = (xa - xhi.astype(f32)).astype(bf16)
    yhi = ya.astype(bf16)
    ylo = (ya - yhi.astype(f32)).astype(bf16)
    xs = jnp.concatenate([xhi, xhi, xlo], axis=-1)
    ys = jnp.concatenate([yhi, ylo, yhi], axis=-1)

    TM = 4096
    grid = (B, M // TM)

    out = pl.pallas_call(
        functools.partial(
            _chamfer_tile_kernel, inv_bn=1.0 / (B * N), inv_bm=1.0 / (B * M)
        ),
        grid=grid,
        in_specs=[
            pl.BlockSpec((1, N, 24), lambda b, m: (b, 0, 0)),
            pl.BlockSpec((1, TM, 24), lambda b, m: (b, m, 0)),
        ],
        out_specs=pl.BlockSpec(
            (1, 1, 1), lambda b, m: (b, 0, 0), memory_space=pltpu.SMEM
        ),
        out_shape=jax.ShapeDtypeStruct((B, 1, 1), f32),
        scratch_shapes=[pltpu.VMEM((N, 1), f32)],
        compiler_params=pltpu.CompilerParams(
            dimension_semantics=("parallel", "arbitrary")
        ),
    )(xs, ys)
    return jnp.sum(out)


# speedup vs baseline: 2.1908x; 1.0015x over previous
import functools

import jax
import jax.numpy as jnp
from jax.experimental import pallas as pl
from jax.experimental.pallas import tpu as pltpu


def _chamfer_tile_kernel(xa_ref, ya_ref, out_ref, rowmin_ref, *, inv_bn, inv_bm):
    m = pl.program_id(1)
    nm = pl.num_programs(1)

    xa = xa_ref[0]
    ya = ya_ref[0]
    d = jax.lax.dot_general(
        xa, ya, (((1,), (1,)), ((), ())), preferred_element_type=jnp.float32
    )

    tile_rowmin = jnp.min(d, axis=1, keepdims=True)

    @pl.when(m == 0)
    def _():
        rowmin_ref[...] = tile_rowmin

    @pl.when(m > 0)
    def _():
        rowmin_ref[...] = jnp.minimum(rowmin_ref[...], tile_rowmin)

    colmin = jnp.min(d, axis=0)
    yx_part = jnp.sum(jnp.maximum(colmin, 0.0)) * inv_bm

    @pl.when(m == 0)
    def _():
        out_ref[0, 0, 0] = 0.0

    out_ref[0, 0, 0] += yx_part

    @pl.when(m == nm - 1)
    def _():
        xy_sum = jnp.sum(jnp.maximum(rowmin_ref[...], 0.0))
        out_ref[0, 0, 0] += xy_sum * inv_bn


@jax.jit
def kernel(x, y):
    B, N, D = x.shape
    _, M, _ = y.shape
    f32 = jnp.float32

    x = x.astype(f32)
    y = y.astype(f32)
    x2 = jnp.sum(x * x, axis=-1, keepdims=True)
    y2 = jnp.sum(y * y, axis=-1, keepdims=True)
    ones_x = jnp.ones_like(x2)
    ones_y = jnp.ones_like(y2)
    zpad_x = jnp.zeros((B, N, 3), f32)
    zpad_y = jnp.zeros((B, M, 3), f32)
    xa = jnp.concatenate([-2.0 * x, x2, ones_x, zpad_x], axis=-1)
    ya = jnp.concatenate([y, ones_y, y2, zpad_y], axis=-1)

    bf16 = jnp.bfloat16
    xhi = xa.astype(bf16)
    xlo = (xa - xhi.astype(f32)).astype(bf16)
    yhi = ya.astype(bf16)
    ylo = (ya - yhi.astype(f32)).astype(bf16)
    xs = jnp.concatenate([xhi, xhi, xlo], axis=-1)
    ys = jnp.concatenate([yhi, ylo, yhi], axis=-1)

    TM = 4096
    grid = (B, M // TM)

    out = pl.pallas_call(
        functools.partial(
            _chamfer_tile_kernel, inv_bn=1.0 / (B * N), inv_bm=1.0 / (B * M)
        ),
        grid=grid,
        in_specs=[
            pl.BlockSpec((1, N, 24), lambda b, m: (b, 0, 0)),
            pl.BlockSpec((1, TM, 24), lambda b, m: (b, m, 0)),
        ],
        out_specs=pl.BlockSpec(
            (1, 1, 1), lambda b, m: (b, 0, 0), memory_space=pltpu.SMEM
        ),
        out_shape=jax.ShapeDtypeStruct((B, 1, 1), f32),
        scratch_shapes=[pltpu.VMEM((N, 1), f32)],
        compiler_params=pltpu.CompilerParams(
            dimension_semantics=("parallel", "arbitrary")
        ),
    )(xs, ys)
    return jnp.sum(out)
